# baseline (speedup 1.0000x reference)
import functools

import jax
import jax.numpy as jnp
from jax import lax
from jax.experimental import pallas as pl
from jax.experimental.pallas import tpu as pltpu
from jax.experimental.pallas import tpu_sc as plsc

_N = 10000
_E = 320000
_DF = 128
_DE = 16
_DO = 128

_NC = 2
_NS = 16
_NW = _NC * _NS
_CHUNK = 128
_NCH = _E // _CHUNK
_RPT = _N // 10

_mesh = plsc.VectorSubcoreMesh(core_axis_name="c", subcore_axis_name="s")


def _tile_chunks(wid):
    nch = jnp.where(wid < 4, 79, 78)
    cstart = jnp.where(wid < 4, 79 * wid, 78 * wid + 4)
    return nch, cstart


def _edge_stats_body(ei_h, ea_h, cnt_o, b_o,
                     eib0, eib1, ebuf, wide0, wide1,
                     sidx0, sidx1, dsh0, dsh1, dix0, dix1,
                     ones, z1, zw, cbuf, cnt_sd, acc_b,
                     lsem0, lsem1, easem, csem0, csem1, wsem0, wsem1):
    c = lax.axis_index("c")
    s = lax.axis_index("s")
    wid = c * _NS + s
    nch, cstart = _tile_chunks(wid)

    eib = (eib0, eib1)
    wide = (wide0, wide1)
    sidx = (sidx0, sidx1)
    dsh = (dsh0, dsh1)
    dix = (dix0, dix1)
    lsem = (lsem0, lsem1)
    csem = (csem0, csem1)
    wsem = (wsem0, wsem1)

    for j in range(_CHUNK // 16):
        ones[pl.ds(j * 16, 16)] = jnp.ones((16,), jnp.float32)

    def zfill1(i, _):
        z1[pl.ds(i * 16, 16)] = jnp.zeros((16,), jnp.float32)
        return 0
    lax.fori_loop(0, 2000 // 16, zfill1, 0)

    def zfillw(i, _):
        for j in range(_DF // 16):
            zw[i, pl.ds(j * 16, 16)] = jnp.zeros((16,), jnp.float32)
        return 0
    lax.fori_loop(0, 40, zfillw, 0)

    for h in range(2):
        def zfill_wide(i, _):
            for j in range(_DF // 16):
                wide[h][i, pl.ds(j * 16, 16)] = jnp.zeros((16,), jnp.float32)
            return 0
        lax.fori_loop(0, _CHUNK // 2, zfill_wide, 0)

    @pl.when(s < 10)
    def _():
        pltpu.sync_copy(z1, cnt_sd.at[pl.ds(s * 2000, 2000)])
        for k in range(25):
            pltpu.sync_copy(zw, acc_b.at[pl.ds(s * _RPT + k * 40, 40)])
    plsc.subcore_barrier()

    def ei_load(ci, b):
        eoff = (cstart + ci) * _CHUNK
        pltpu.async_copy(ei_h.at[:, pl.ds(eoff, _CHUNK)], eib[b], lsem[b])

    def ei_wait(ci, b):
        eoff = (cstart + ci) * _CHUNK
        pltpu.make_async_copy(ei_h.at[:, pl.ds(eoff, _CHUNK)],
                              eib[b], lsem[b]).wait()

    def ea_load(ci):
        eoff = (cstart + ci) * _CHUNK
        pltpu.async_copy(ea_h.at[pl.ds(eoff, _CHUNK)], ebuf, easem)

    def ea_wait(ci):
        eoff = (cstart + ci) * _CHUNK
        pltpu.make_async_copy(ea_h.at[pl.ds(eoff, _CHUNK)], ebuf, easem).wait()

    def counts_issue(b):
        pltpu.async_copy(ones, cnt_sd.at[sidx[b]], csem[b], add=True)
        pltpu.async_copy(ones, cnt_sd.at[dsh[b]], csem[b], add=True)

    def counts_drain(b):
        pltpu.make_async_copy(ones, cnt_sd.at[sidx[b]], csem[b]).wait()
        pltpu.make_async_copy(ones, cnt_sd.at[dsh[b]], csem[b]).wait()

    def wide_issue(h, b):
        for e in range(_CHUNK // 2):
            wide[h][e, pl.ds(0, 16)] = ebuf[64 * h + e, pl.ds(0, 16)]
        for j in range(4):
            sl = pl.ds(j * 16, 16)
            dix[h][sl] = eib[b][1, pl.ds(64 * h + j * 16, 16)]
        pltpu.async_copy(wide[h], acc_b.at[dix[h]], wsem[h], add=True)

    def wide_drain(h):
        pltpu.make_async_copy(wide[h], acc_b.at[dix[h]], wsem[h]).wait()

    def stage_counts(b):
        for j in range(_CHUNK // 16):
            sl = pl.ds(j * 16, 16)
            sidx[b][sl] = eib[b][0, sl]
            dsh[b][sl] = eib[b][1, sl] + _N

    ei_load(0, 0)
    ea_load(0)

    def step(j, _):
        cA, cB, cC = 2 * j, 2 * j + 1, 2 * j + 2
        @pl.when(j > 0)
        def _():
            counts_drain(0)
        ei_load(cB, 1)
        ei_wait(cA, 0)
        ea_wait(cA)
        @pl.when(j > 0)
        def _():
            wide_drain(0)
        wide_issue(0, 0)
        @pl.when(j > 0)
        def _():
            wide_drain(1)
        wide_issue(1, 0)
        ea_load(cB)
        stage_counts(0)
        counts_issue(0)
        @pl.when(j > 0)
        def _():
            counts_drain(1)
        ei_wait(cB, 1)
        ea_wait(cB)
        wide_drain(0)
        wide_issue(0, 1)
        wide_drain(1)
        wide_issue(1, 1)
        @pl.when(cC < nch)
        def _():
            ei_load(cC, 0)
            ea_load(cC)
        stage_counts(1)
        counts_issue(1)
        return 0
    lax.fori_loop(0, 39, step, 0)
    counts_drain(0)
    counts_drain(1)

    @pl.when(nch == 79)
    def _():
        ei_wait(78, 0)
        ea_wait(78)
        wide_drain(0)
        wide_issue(0, 0)
        wide_drain(1)
        wide_issue(1, 0)
        stage_counts(0)
        counts_issue(0)
        counts_drain(0)
        wide_drain(0)
        wide_drain(1)

    @pl.when(nch == 78)
    def _():
        wide_drain(0)
        wide_drain(1)
    plsc.subcore_barrier()

    @pl.when(s < 10)
    def _():
        pltpu.sync_copy(cnt_sd.at[pl.ds(s * 2000, 2000)], cbuf)
        pltpu.sync_copy(cbuf, cnt_o.at[pl.ds(c * 2 * _N + s * 2000, 2000)])
        pltpu.sync_copy(acc_b.at[pl.ds(s * _RPT, _RPT)],
                        b_o.at[c, pl.ds(s * _RPT, _RPT)])


_edge_stats = pl.kernel(
    _edge_stats_body,
    out_type=[jax.ShapeDtypeStruct((_NC * 2 * _N,), jnp.float32),
              jax.ShapeDtypeStruct((_NC, _N, _DF), jnp.float32)],
    mesh=_mesh,
    scratch_types=[
        pltpu.VMEM((2, _CHUNK), jnp.int32),
        pltpu.VMEM((2, _CHUNK), jnp.int32),
        pltpu.VMEM((_CHUNK, _DE), jnp.float32),
        pltpu.VMEM((_CHUNK // 2, _DF), jnp.float32),
        pltpu.VMEM((_CHUNK // 2, _DF), jnp.float32),
        pltpu.VMEM((_CHUNK,), jnp.int32),
        pltpu.VMEM((_CHUNK,), jnp.int32),
        pltpu.VMEM((_CHUNK,), jnp.int32),
        pltpu.VMEM((_CHUNK,), jnp.int32),
        pltpu.VMEM((_CHUNK // 2,), jnp.int32),
        pltpu.VMEM((_CHUNK // 2,), jnp.int32),
        pltpu.VMEM((_CHUNK,), jnp.float32),
        pltpu.VMEM((2000,), jnp.float32),
        pltpu.VMEM((40, _DF), jnp.float32),
        pltpu.VMEM((2000,), jnp.float32),
        pltpu.VMEM_SHARED((2 * _N,), jnp.float32),
        pltpu.VMEM_SHARED((_N, _DF), jnp.float32),
        pltpu.SemaphoreType.DMA,
        pltpu.SemaphoreType.DMA,
        pltpu.SemaphoreType.DMA,
        pltpu.SemaphoreType.DMA,
        pltpu.SemaphoreType.DMA,
        pltpu.SemaphoreType.DMA,
        pltpu.SemaphoreType.DMA,
    ],
)


def _aggregate_body(ei_h, feat_h, a_o,
                    eib0, eib1, didx0, didx1, rows0, rows1, zrow, acc_a,
                    lsem0, lsem1, gsem0, gsem1, ssem0, ssem1):
    c = lax.axis_index("c")
    s = lax.axis_index("s")
    wid = c * _NS + s
    nch, cstart = _tile_chunks(wid)

    eib = (eib0, eib1)
    didx = (didx0, didx1)
    rows = (rows0, rows1)
    lsem = (lsem0, lsem1)
    gsem = (gsem0, gsem1)
    ssem = (ssem0, ssem1)

    def zfill(i, _):
        for j in range(_DF // 16):
            zrow[i, pl.ds(j * 16, 16)] = jnp.zeros((16,), jnp.float32)
        return 0
    lax.fori_loop(0, 40, zfill, 0)

    @pl.when(s < 10)
    def _():
        for k in range(25):
            pltpu.sync_copy(zrow, acc_a.at[pl.ds(s * _RPT + k * 40, 40)])
    plsc.subcore_barrier()

    def load(ci, b):
        eoff = (cstart + ci) * _CHUNK
        pltpu.async_copy(ei_h.at[:, pl.ds(eoff, _CHUNK)], eib[b], lsem[b])

    def wait_load(ci, b):
        eoff = (cstart + ci) * _CHUNK
        pltpu.make_async_copy(ei_h.at[:, pl.ds(eoff, _CHUNK)],
                              eib[b], lsem[b]).wait()

    def gather(b):
        pltpu.async_copy(feat_h.at[eib[b].at[0]], rows[b], gsem[b])

    def wait_gather(b):
        pltpu.make_async_copy(feat_h.at[eib[b].at[0]], rows[b], gsem[b]).wait()

    def scatter(b):
        for j in range(_CHUNK // 16):
            sl = pl.ds(j * 16, 16)
            didx[b][sl] = eib[b][1, sl]
        pltpu.async_copy(rows[b], acc_a.at[didx[b]], ssem[b], add=True)

    def drain(b):
        pltpu.make_async_copy(rows[b], acc_a.at[didx[b]], ssem[b]).wait()

    load(0, 0)
    load(1, 1)
    wait_load(0, 0)
    gather(0)

    def step(j, _):
        cB, cC, cD = 2 * j + 1, 2 * j + 2, 2 * j + 3
        @pl.when(j > 0)
        def _():
            drain(1)
        wait_load(cB, 1)
        gather(1)
        wait_gather(0)
        scatter(0)
        @pl.when(cC < nch)
        def _():
            load(cC, 0)
        drain(0)
        @pl.when(cC < nch)
        def _():
            wait_load(cC, 0)
            gather(0)
        wait_gather(1)
        scatter(1)
        @pl.when(cD < nch)
        def _():
            load(cD, 1)
        return 0
    lax.fori_loop(0, 39, step, 0)
    drain(1)

    @pl.when(nch == 79)
    def _():
        wait_gather(0)
        scatter(0)
        drain(0)
    plsc.subcore_barrier()

    @pl.when(s < 10)
    def _():
        pltpu.sync_copy(acc_a.at[pl.ds(s * _RPT, _RPT)],
                        a_o.at[c, pl.ds(s * _RPT, _RPT)])


_aggregate = pl.kernel(
    _aggregate_body,
    out_type=jax.ShapeDtypeStruct((_NC, _N, _DF), jnp.float32),
    mesh=_mesh,
    scratch_types=[
        pltpu.VMEM((2, _CHUNK), jnp.int32),
        pltpu.VMEM((2, _CHUNK), jnp.int32),
        pltpu.VMEM((_CHUNK,), jnp.int32),
        pltpu.VMEM((_CHUNK,), jnp.int32),
        pltpu.VMEM((_CHUNK, _DF), jnp.float32),
        pltpu.VMEM((_CHUNK, _DF), jnp.float32),
        pltpu.VMEM((40, _DF), jnp.float32),
        pltpu.VMEM_SHARED((_N, _DF), jnp.float32),
        pltpu.SemaphoreType.DMA,
        pltpu.SemaphoreType.DMA,
        pltpu.SemaphoreType.DMA,
        pltpu.SemaphoreType.DMA,
        pltpu.SemaphoreType.DMA,
        pltpu.SemaphoreType.DMA,
    ],
)


_BS = 2000
_NB = _N // _BS


def _scale_body(x_ref, c0_ref, c1_ref, feat_ref):
    deg = c0_ref[...] + c1_ref[...]
    ns = lax.rsqrt(jnp.maximum(deg, 1.0))
    feat_ref[...] = x_ref[...] * ns


def _scale(x, c0, c1):
    return pl.pallas_call(
        _scale_body,
        grid=(_NB,),
        in_specs=[pl.BlockSpec((_BS, _DF), lambda i: (i, 0)),
                  pl.BlockSpec((_BS, 1), lambda i: (i, 0)),
                  pl.BlockSpec((_BS, 1), lambda i: (i, 0))],
        out_specs=pl.BlockSpec((_BS, _DF), lambda i: (i, 0)),
        out_shape=jax.ShapeDtypeStruct((_N, _DF), jnp.float32),
    )(x, c0, c1)


def _final_body(a_ref, b_ref, w_ref, bias_ref, d0_ref, d1_ref, o_ref):
    a = a_ref[0] + a_ref[1]
    b = (b_ref[0] + b_ref[1])[:, :_DE]
    w = w_ref[...]
    h = jnp.dot(a, w[:_DF], preferred_element_type=jnp.float32,
                precision=lax.Precision.HIGHEST)
    h = h + jnp.dot(b, w[_DF:], preferred_element_type=jnp.float32,
                    precision=lax.Precision.HIGHEST)
    deg = d0_ref[...] + d1_ref[...]
    nd = lax.rsqrt(jnp.maximum(deg, 1.0))
    o_ref[...] = h * nd + bias_ref[...]


def _final(a, b, w, bias, d0, d1):
    return pl.pallas_call(
        _final_body,
        grid=(_NB,),
        in_specs=[pl.BlockSpec((_NC, _BS, _DF), lambda i: (0, i, 0)),
                  pl.BlockSpec((_NC, _BS, _DF), lambda i: (0, i, 0)),
                  pl.BlockSpec((_DF + _DE, _DO), lambda i: (0, 0)),
                  pl.BlockSpec((_DO,), lambda i: (0,)),
                  pl.BlockSpec((_BS, 1), lambda i: (i, 0)),
                  pl.BlockSpec((_BS, 1), lambda i: (i, 0))],
        out_specs=pl.BlockSpec((_BS, _DO), lambda i: (i, 0)),
        out_shape=jax.ShapeDtypeStruct((_N, _DO), jnp.float32),
    )(a, b, w, bias, d0, d1)


def kernel(x, edge_index, edge_attr, weight, bias):
    cnt, b_part = _edge_stats(edge_index, edge_attr)
    cs0 = cnt[0:_N].reshape(_N, 1)
    cd0 = cnt[_N:2 * _N].reshape(_N, 1)
    cs1 = cnt[2 * _N:3 * _N].reshape(_N, 1)
    cd1 = cnt[3 * _N:4 * _N].reshape(_N, 1)
    feat = _scale(x, cs0, cs1)
    a_part = _aggregate(edge_index, feat)
    return _final(a_part, b_part, weight, bias, cd0, cd1)


# speedup vs baseline: 1.0091x; 1.0091x over previous
import functools

import jax
import jax.numpy as jnp
from jax import lax
from jax.experimental import pallas as pl
from jax.experimental.pallas import tpu as pltpu
from jax.experimental.pallas import tpu_sc as plsc

_N = 10000
_E = 320000
_DF = 128
_DE = 16
_DO = 128

_NC = 2
_NS = 16
_NW = _NC * _NS
_CHUNK = 128
_NCH = _E // _CHUNK
_RPT = _N // 10

_mesh = plsc.VectorSubcoreMesh(core_axis_name="c", subcore_axis_name="s")


def _tile_chunks(wid):
    nch = jnp.where(wid < 4, 79, 78)
    cstart = jnp.where(wid < 4, 79 * wid, 78 * wid + 4)
    return nch, cstart


def _edge_stats_body(ei_h, eaf_h, cnt_o, b_o,
                     eib0, eib1, ebuf, wide0, wide1,
                     sidx0, sidx1, dsh0, dsh1, dix0, dix1,
                     ones, z1, zw, cbuf, cnt_sd, acc_b,
                     lsem0, lsem1, easem, csem0, csem1, wsem0, wsem1):
    c = lax.axis_index("c")
    s = lax.axis_index("s")
    wid = c * _NS + s
    nch, cstart = _tile_chunks(wid)

    eib = (eib0, eib1)
    wide = (wide0, wide1)
    sidx = (sidx0, sidx1)
    dsh = (dsh0, dsh1)
    dix = (dix0, dix1)
    lsem = (lsem0, lsem1)
    csem = (csem0, csem1)
    wsem = (wsem0, wsem1)

    for j in range(_CHUNK // 16):
        ones[pl.ds(j * 16, 16)] = jnp.ones((16,), jnp.float32)

    def zfill1(i, _):
        z1[pl.ds(i * 16, 16)] = jnp.zeros((16,), jnp.float32)
        return 0
    lax.fori_loop(0, 2000 // 16, zfill1, 0)

    def zfillw(i, _):
        for j in range(_DF // 16):
            zw[i, pl.ds(j * 16, 16)] = jnp.zeros((16,), jnp.float32)
        return 0
    lax.fori_loop(0, 40, zfillw, 0)

    for h in range(2):
        def zfill_wide(i, _):
            for j in range(_DF // 16):
                wide[h][i, pl.ds(j * 16, 16)] = jnp.zeros((16,), jnp.float32)
            return 0
        lax.fori_loop(0, _CHUNK // 2, zfill_wide, 0)

    @pl.when(s < 10)
    def _():
        pltpu.sync_copy(z1, cnt_sd.at[pl.ds(s * 2000, 2000)])
        for k in range(25):
            pltpu.sync_copy(zw, acc_b.at[pl.ds(s * _RPT + k * 40, 40)])
    plsc.subcore_barrier()

    def ei_load(ci, b):
        eoff = (cstart + ci) * _CHUNK
        pltpu.async_copy(ei_h.at[:, pl.ds(eoff, _CHUNK)], eib[b], lsem[b])

    def ei_wait(ci, b):
        eoff = (cstart + ci) * _CHUNK
        pltpu.make_async_copy(ei_h.at[:, pl.ds(eoff, _CHUNK)],
                              eib[b], lsem[b]).wait()

    def ea_load(ci):
        eoff = (cstart + ci) * _CHUNK * _DE
        pltpu.async_copy(eaf_h.at[pl.ds(eoff, _CHUNK * _DE)], ebuf, easem)

    def ea_wait(ci):
        eoff = (cstart + ci) * _CHUNK * _DE
        pltpu.make_async_copy(eaf_h.at[pl.ds(eoff, _CHUNK * _DE)], ebuf, easem).wait()

    def counts_issue(b):
        pltpu.async_copy(ones, cnt_sd.at[sidx[b]], csem[b], add=True)
        pltpu.async_copy(ones, cnt_sd.at[dsh[b]], csem[b], add=True)

    def counts_drain(b):
        pltpu.make_async_copy(ones, cnt_sd.at[sidx[b]], csem[b]).wait()
        pltpu.make_async_copy(ones, cnt_sd.at[dsh[b]], csem[b]).wait()

    def wide_issue(h, b):
        for e in range(_CHUNK // 2):
            wide[h][e, pl.ds(0, 16)] = ebuf[pl.ds((64 * h + e) * _DE, 16)]
        for j in range(4):
            sl = pl.ds(j * 16, 16)
            dix[h][sl] = eib[b][1, pl.ds(64 * h + j * 16, 16)]
        pltpu.async_copy(wide[h], acc_b.at[dix[h]], wsem[h], add=True)

    def wide_drain(h):
        pltpu.make_async_copy(wide[h], acc_b.at[dix[h]], wsem[h]).wait()

    def stage_counts(b):
        for j in range(_CHUNK // 16):
            sl = pl.ds(j * 16, 16)
            sidx[b][sl] = eib[b][0, sl]
            dsh[b][sl] = eib[b][1, sl] + _N

    ei_load(0, 0)
    ea_load(0)

    def step(j, _):
        cA, cB, cC = 2 * j, 2 * j + 1, 2 * j + 2
        @pl.when(j > 0)
        def _():
            counts_drain(0)
        ei_load(cB, 1)
        ei_wait(cA, 0)
        ea_wait(cA)
        @pl.when(j > 0)
        def _():
            wide_drain(0)
        wide_issue(0, 0)
        @pl.when(j > 0)
        def _():
            wide_drain(1)
        wide_issue(1, 0)
        ea_load(cB)
        stage_counts(0)
        counts_issue(0)
        @pl.when(j > 0)
        def _():
            counts_drain(1)
        ei_wait(cB, 1)
        ea_wait(cB)
        wide_drain(0)
        wide_issue(0, 1)
        wide_drain(1)
        wide_issue(1, 1)
        @pl.when(cC < nch)
        def _():
            ei_load(cC, 0)
            ea_load(cC)
        stage_counts(1)
        counts_issue(1)
        return 0
    lax.fori_loop(0, 39, step, 0)
    counts_drain(0)
    counts_drain(1)

    @pl.when(nch == 79)
    def _():
        ei_wait(78, 0)
        ea_wait(78)
        wide_drain(0)
        wide_issue(0, 0)
        wide_drain(1)
        wide_issue(1, 0)
        stage_counts(0)
        counts_issue(0)
        counts_drain(0)
        wide_drain(0)
        wide_drain(1)

    @pl.when(nch == 78)
    def _():
        wide_drain(0)
        wide_drain(1)
    plsc.subcore_barrier()

    @pl.when(s < 10)
    def _():
        pltpu.sync_copy(cnt_sd.at[pl.ds(s * 2000, 2000)], cbuf)
        pltpu.sync_copy(cbuf, cnt_o.at[pl.ds(c * 2 * _N + s * 2000, 2000)])
        pltpu.sync_copy(acc_b.at[pl.ds(s * _RPT, _RPT)],
                        b_o.at[c, pl.ds(s * _RPT, _RPT)])


_edge_stats = pl.kernel(
    _edge_stats_body,
    out_type=[jax.ShapeDtypeStruct((_NC * 2 * _N,), jnp.float32),
              jax.ShapeDtypeStruct((_NC, _N, _DF), jnp.float32)],
    mesh=_mesh,
    scratch_types=[
        pltpu.VMEM((2, _CHUNK), jnp.int32),
        pltpu.VMEM((2, _CHUNK), jnp.int32),
        pltpu.VMEM((_CHUNK * _DE,), jnp.float32),
        pltpu.VMEM((_CHUNK // 2, _DF), jnp.float32),
        pltpu.VMEM((_CHUNK // 2, _DF), jnp.float32),
        pltpu.VMEM((_CHUNK,), jnp.int32),
        pltpu.VMEM((_CHUNK,), jnp.int32),
        pltpu.VMEM((_CHUNK,), jnp.int32),
        pltpu.VMEM((_CHUNK,), jnp.int32),
        pltpu.VMEM((_CHUNK // 2,), jnp.int32),
        pltpu.VMEM((_CHUNK // 2,), jnp.int32),
        pltpu.VMEM((_CHUNK,), jnp.float32),
        pltpu.VMEM((2000,), jnp.float32),
        pltpu.VMEM((40, _DF), jnp.float32),
        pltpu.VMEM((2000,), jnp.float32),
        pltpu.VMEM_SHARED((2 * _N,), jnp.float32),
        pltpu.VMEM_SHARED((_N, _DF), jnp.float32),
        pltpu.SemaphoreType.DMA,
        pltpu.SemaphoreType.DMA,
        pltpu.SemaphoreType.DMA,
        pltpu.SemaphoreType.DMA,
        pltpu.SemaphoreType.DMA,
        pltpu.SemaphoreType.DMA,
        pltpu.SemaphoreType.DMA,
    ],
)


def _aggregate_body(ei_h, feat_h, a_o,
                    eib0, eib1, didx0, didx1, rows0, rows1, zrow, acc_a,
                    lsem0, lsem1, gsem0, gsem1, ssem0, ssem1):
    c = lax.axis_index("c")
    s = lax.axis_index("s")
    wid = c * _NS + s
    nch, cstart = _tile_chunks(wid)

    eib = (eib0, eib1)
    didx = (didx0, didx1)
    rows = (rows0, rows1)
    lsem = (lsem0, lsem1)
    gsem = (gsem0, gsem1)
    ssem = (ssem0, ssem1)

    def zfill(i, _):
        for j in range(_DF // 16):
            zrow[i, pl.ds(j * 16, 16)] = jnp.zeros((16,), jnp.float32)
        return 0
    lax.fori_loop(0, 40, zfill, 0)

    @pl.when(s < 10)
    def _():
        for k in range(25):
            pltpu.sync_copy(zrow, acc_a.at[pl.ds(s * _RPT + k * 40, 40)])
    plsc.subcore_barrier()

    def load(ci, b):
        eoff = (cstart + ci) * _CHUNK
        pltpu.async_copy(ei_h.at[:, pl.ds(eoff, _CHUNK)], eib[b], lsem[b])

    def wait_load(ci, b):
        eoff = (cstart + ci) * _CHUNK
        pltpu.make_async_copy(ei_h.at[:, pl.ds(eoff, _CHUNK)],
                              eib[b], lsem[b]).wait()

    def gather(b):
        pltpu.async_copy(feat_h.at[eib[b].at[0]], rows[b], gsem[b])

    def wait_gather(b):
        pltpu.make_async_copy(feat_h.at[eib[b].at[0]], rows[b], gsem[b]).wait()

    def scatter(b):
        for j in range(_CHUNK // 16):
            sl = pl.ds(j * 16, 16)
            didx[b][sl] = eib[b][1, sl]
        pltpu.async_copy(rows[b], acc_a.at[didx[b]], ssem[b], add=True)

    def drain(b):
        pltpu.make_async_copy(rows[b], acc_a.at[didx[b]], ssem[b]).wait()

    load(0, 0)
    load(1, 1)
    wait_load(0, 0)
    gather(0)

    def step(j, _):
        cB, cC, cD = 2 * j + 1, 2 * j + 2, 2 * j + 3
        @pl.when(j > 0)
        def _():
            drain(1)
        wait_load(cB, 1)
        gather(1)
        wait_gather(0)
        scatter(0)
        @pl.when(cC < nch)
        def _():
            load(cC, 0)
        drain(0)
        @pl.when(cC < nch)
        def _():
            wait_load(cC, 0)
            gather(0)
        wait_gather(1)
        scatter(1)
        @pl.when(cD < nch)
        def _():
            load(cD, 1)
        return 0
    lax.fori_loop(0, 39, step, 0)
    drain(1)

    @pl.when(nch == 79)
    def _():
        wait_gather(0)
        scatter(0)
        drain(0)
    plsc.subcore_barrier()

    @pl.when(s < 10)
    def _():
        pltpu.sync_copy(acc_a.at[pl.ds(s * _RPT, _RPT)],
                        a_o.at[c, pl.ds(s * _RPT, _RPT)])


_aggregate = pl.kernel(
    _aggregate_body,
    out_type=jax.ShapeDtypeStruct((_NC, _N, _DF), jnp.float32),
    mesh=_mesh,
    scratch_types=[
        pltpu.VMEM((2, _CHUNK), jnp.int32),
        pltpu.VMEM((2, _CHUNK), jnp.int32),
        pltpu.VMEM((_CHUNK,), jnp.int32),
        pltpu.VMEM((_CHUNK,), jnp.int32),
        pltpu.VMEM((_CHUNK, _DF), jnp.float32),
        pltpu.VMEM((_CHUNK, _DF), jnp.float32),
        pltpu.VMEM((40, _DF), jnp.float32),
        pltpu.VMEM_SHARED((_N, _DF), jnp.float32),
        pltpu.SemaphoreType.DMA,
        pltpu.SemaphoreType.DMA,
        pltpu.SemaphoreType.DMA,
        pltpu.SemaphoreType.DMA,
        pltpu.SemaphoreType.DMA,
        pltpu.SemaphoreType.DMA,
    ],
)


_BS = 2000
_NB = _N // _BS


def _scale_body(x_ref, c0_ref, c1_ref, feat_ref):
    deg = c0_ref[...] + c1_ref[...]
    ns = lax.rsqrt(jnp.maximum(deg, 1.0))
    feat_ref[...] = x_ref[...] * ns


def _scale(x, c0, c1):
    return pl.pallas_call(
        _scale_body,
        grid=(_NB,),
        in_specs=[pl.BlockSpec((_BS, _DF), lambda i: (i, 0)),
                  pl.BlockSpec((_BS, 1), lambda i: (i, 0)),
                  pl.BlockSpec((_BS, 1), lambda i: (2 * _N // _BS + i, 0))],
        out_specs=pl.BlockSpec((_BS, _DF), lambda i: (i, 0)),
        out_shape=jax.ShapeDtypeStruct((_N, _DF), jnp.float32),
    )(x, c0, c1)


def _final_body(a_ref, b_ref, w_ref, bias_ref, d0_ref, d1_ref, o_ref):
    a = a_ref[0] + a_ref[1]
    b = (b_ref[0] + b_ref[1])[:, :_DE]
    w = w_ref[...]
    h = jnp.dot(a, w[:_DF], preferred_element_type=jnp.float32,
                precision=lax.Precision.HIGHEST)
    h = h + jnp.dot(b, w[_DF:], preferred_element_type=jnp.float32,
                    precision=lax.Precision.HIGHEST)
    deg = d0_ref[...] + d1_ref[...]
    nd = lax.rsqrt(jnp.maximum(deg, 1.0))
    o_ref[...] = h * nd + bias_ref[...]


def _final(a, b, w, bias, d0, d1):
    return pl.pallas_call(
        _final_body,
        grid=(_NB,),
        in_specs=[pl.BlockSpec((_NC, _BS, _DF), lambda i: (0, i, 0)),
                  pl.BlockSpec((_NC, _BS, _DF), lambda i: (0, i, 0)),
                  pl.BlockSpec((_DF + _DE, _DO), lambda i: (0, 0)),
                  pl.BlockSpec((_DO,), lambda i: (0,)),
                  pl.BlockSpec((_BS, 1), lambda i: (_N // _BS + i, 0)),
                  pl.BlockSpec((_BS, 1), lambda i: (3 * _N // _BS + i, 0))],
        out_specs=pl.BlockSpec((_BS, _DO), lambda i: (i, 0)),
        out_shape=jax.ShapeDtypeStruct((_N, _DO), jnp.float32),
    )(a, b, w, bias, d0, d1)


def kernel(x, edge_index, edge_attr, weight, bias):
    ea_flat = edge_attr.reshape(-1)
    cnt, b_part = _edge_stats(edge_index, ea_flat)
    cnt4 = cnt.reshape(4 * _N, 1)
    feat = _scale(x, cnt4, cnt4)
    a_part = _aggregate(edge_index, feat)
    return _final(a_part, b_part, weight, bias, cnt4, cnt4)


# speedup vs baseline: 1.0156x; 1.0064x over previous
import functools

import jax
import jax.numpy as jnp
from jax import lax
from jax.experimental import pallas as pl
from jax.experimental.pallas import tpu as pltpu
from jax.experimental.pallas import tpu_sc as plsc

_N = 10000
_E = 320000
_DF = 128
_DE = 16
_DO = 128

_NC = 2
_NS = 16
_NW = _NC * _NS
_CHUNK = 128
_NCH = _E // _CHUNK
_RPT = _N // 10

_mesh = plsc.VectorSubcoreMesh(core_axis_name="c", subcore_axis_name="s")


def _tile_chunks(wid):
    nch = jnp.where(wid < 4, 79, 78)
    cstart = jnp.where(wid < 4, 79 * wid, 78 * wid + 4)
    return nch, cstart


def _edge_stats_body(ei_h, eaf_h, cnt_o, b_o,
                     eib0, eib1, ebuf, wide0, wide1,
                     sidx0, sidx1, dsh0, dsh1, dix0, dix1,
                     ones, z1, zw, cbuf, cnt_sd, acc_b,
                     lsem0, lsem1, easem, csem0, csem1, wsem0, wsem1):
    c = lax.axis_index("c")
    s = lax.axis_index("s")
    wid = c * _NS + s
    nch, cstart = _tile_chunks(wid)

    eib = (eib0, eib1)
    wide = (wide0, wide1)
    sidx = (sidx0, sidx1)
    dsh = (dsh0, dsh1)
    dix = (dix0, dix1)
    lsem = (lsem0, lsem1)
    csem = (csem0, csem1)
    wsem = (wsem0, wsem1)

    for j in range(_CHUNK // 16):
        ones[pl.ds(j * 16, 16)] = jnp.ones((16,), jnp.float32)

    def zfill1(i, _):
        z1[pl.ds(i * 16, 16)] = jnp.zeros((16,), jnp.float32)
        return 0
    lax.fori_loop(0, 2000 // 16, zfill1, 0)

    def zfillw(i, _):
        for j in range(_DF // 16):
            zw[i, pl.ds(j * 16, 16)] = jnp.zeros((16,), jnp.float32)
        return 0
    lax.fori_loop(0, 40, zfillw, 0)

    for h in range(2):
        def zfill_wide(i, _):
            for j in range(_DF // 16):
                wide[h][i, pl.ds(j * 16, 16)] = jnp.zeros((16,), jnp.float32)
            return 0
        lax.fori_loop(0, _CHUNK // 2, zfill_wide, 0)

    @pl.when(s < 10)
    def _():
        pltpu.async_copy(z1, cnt_sd.at[pl.ds(s * 2000, 2000)], easem)
        for k in range(25):
            pltpu.async_copy(zw, acc_b.at[pl.ds(s * _RPT + k * 40, 40)], easem)
        pltpu.make_async_copy(z1, cnt_sd.at[pl.ds(s * 2000, 2000)], easem).wait()
        for k in range(25):
            pltpu.make_async_copy(zw, acc_b.at[pl.ds(s * _RPT + k * 40, 40)],
                                  easem).wait()
    plsc.subcore_barrier()

    def ei_load(ci, b):
        eoff = (cstart + ci) * _CHUNK
        pltpu.async_copy(ei_h.at[:, pl.ds(eoff, _CHUNK)], eib[b], lsem[b])

    def ei_wait(ci, b):
        eoff = (cstart + ci) * _CHUNK
        pltpu.make_async_copy(ei_h.at[:, pl.ds(eoff, _CHUNK)],
                              eib[b], lsem[b]).wait()

    def ea_load(ci):
        eoff = (cstart + ci) * _CHUNK * _DE
        pltpu.async_copy(eaf_h.at[pl.ds(eoff, _CHUNK * _DE)], ebuf, easem)

    def ea_wait(ci):
        eoff = (cstart + ci) * _CHUNK * _DE
        pltpu.make_async_copy(eaf_h.at[pl.ds(eoff, _CHUNK * _DE)], ebuf, easem).wait()

    def counts_issue(b):
        pltpu.async_copy(ones, cnt_sd.at[sidx[b]], csem[b], add=True)
        pltpu.async_copy(ones, cnt_sd.at[dsh[b]], csem[b], add=True)

    def counts_drain(b):
        pltpu.make_async_copy(ones, cnt_sd.at[sidx[b]], csem[b]).wait()
        pltpu.make_async_copy(ones, cnt_sd.at[dsh[b]], csem[b]).wait()

    def wide_issue(h, b):
        for e in range(_CHUNK // 2):
            wide[h][e, pl.ds(0, 16)] = ebuf[pl.ds((64 * h + e) * _DE, 16)]
        for j in range(4):
            sl = pl.ds(j * 16, 16)
            dix[h][sl] = eib[b][1, pl.ds(64 * h + j * 16, 16)]
        pltpu.async_copy(wide[h], acc_b.at[dix[h]], wsem[h], add=True)

    def wide_drain(h):
        pltpu.make_async_copy(wide[h], acc_b.at[dix[h]], wsem[h]).wait()

    def stage_counts(b):
        for j in range(_CHUNK // 16):
            sl = pl.ds(j * 16, 16)
            sidx[b][sl] = eib[b][0, sl]
            dsh[b][sl] = eib[b][1, sl] + _N

    ei_load(0, 0)
    ea_load(0)

    def step(j, _):
        cA, cB, cC = 2 * j, 2 * j + 1, 2 * j + 2
        @pl.when(j > 0)
        def _():
            counts_drain(0)
        ei_load(cB, 1)
        ei_wait(cA, 0)
        ea_wait(cA)
        @pl.when(j > 0)
        def _():
            wide_drain(0)
        wide_issue(0, 0)
        @pl.when(j > 0)
        def _():
            wide_drain(1)
        wide_issue(1, 0)
        ea_load(cB)
        stage_counts(0)
        counts_issue(0)
        @pl.when(j > 0)
        def _():
            counts_drain(1)
        ei_wait(cB, 1)
        ea_wait(cB)
        wide_drain(0)
        wide_issue(0, 1)
        wide_drain(1)
        wide_issue(1, 1)
        @pl.when(cC < nch)
        def _():
            ei_load(cC, 0)
            ea_load(cC)
        stage_counts(1)
        counts_issue(1)
        return 0
    lax.fori_loop(0, 39, step, 0)
    counts_drain(0)
    counts_drain(1)

    @pl.when(nch == 79)
    def _():
        ei_wait(78, 0)
        ea_wait(78)
        wide_drain(0)
        wide_issue(0, 0)
        wide_drain(1)
        wide_issue(1, 0)
        stage_counts(0)
        counts_issue(0)
        counts_drain(0)
        wide_drain(0)
        wide_drain(1)

    @pl.when(nch == 78)
    def _():
        wide_drain(0)
        wide_drain(1)
    plsc.subcore_barrier()

    @pl.when(s < 10)
    def _():
        pltpu.sync_copy(cnt_sd.at[pl.ds(s * 2000, 2000)], cbuf)
        pltpu.sync_copy(cbuf, cnt_o.at[pl.ds(c * 2 * _N + s * 2000, 2000)])
        pltpu.sync_copy(acc_b.at[pl.ds(s * _RPT, _RPT)],
                        b_o.at[c, pl.ds(s * _RPT, _RPT)])


_edge_stats = pl.kernel(
    _edge_stats_body,
    out_type=[jax.ShapeDtypeStruct((_NC * 2 * _N,), jnp.float32),
              jax.ShapeDtypeStruct((_NC, _N, _DF), jnp.float32)],
    mesh=_mesh,
    scratch_types=[
        pltpu.VMEM((2, _CHUNK), jnp.int32),
        pltpu.VMEM((2, _CHUNK), jnp.int32),
        pltpu.VMEM((_CHUNK * _DE,), jnp.float32),
        pltpu.VMEM((_CHUNK // 2, _DF), jnp.float32),
        pltpu.VMEM((_CHUNK // 2, _DF), jnp.float32),
        pltpu.VMEM((_CHUNK,), jnp.int32),
        pltpu.VMEM((_CHUNK,), jnp.int32),
        pltpu.VMEM((_CHUNK,), jnp.int32),
        pltpu.VMEM((_CHUNK,), jnp.int32),
        pltpu.VMEM((_CHUNK // 2,), jnp.int32),
        pltpu.VMEM((_CHUNK // 2,), jnp.int32),
        pltpu.VMEM((_CHUNK,), jnp.float32),
        pltpu.VMEM((2000,), jnp.float32),
        pltpu.VMEM((40, _DF), jnp.float32),
        pltpu.VMEM((2000,), jnp.float32),
        pltpu.VMEM_SHARED((2 * _N,), jnp.float32),
        pltpu.VMEM_SHARED((_N, _DF), jnp.float32),
        pltpu.SemaphoreType.DMA,
        pltpu.SemaphoreType.DMA,
        pltpu.SemaphoreType.DMA,
        pltpu.SemaphoreType.DMA,
        pltpu.SemaphoreType.DMA,
        pltpu.SemaphoreType.DMA,
        pltpu.SemaphoreType.DMA,
    ],
)


def _aggregate_body(ei_h, feat_h, a_o,
                    eib0, eib1, didx0, didx1, rows0, rows1, zrow, acc_a,
                    lsem0, lsem1, gsem0, gsem1, ssem0, ssem1):
    c = lax.axis_index("c")
    s = lax.axis_index("s")
    wid = c * _NS + s
    nch, cstart = _tile_chunks(wid)

    eib = (eib0, eib1)
    didx = (didx0, didx1)
    rows = (rows0, rows1)
    lsem = (lsem0, lsem1)
    gsem = (gsem0, gsem1)
    ssem = (ssem0, ssem1)

    def zfill(i, _):
        for j in range(_DF // 16):
            zrow[i, pl.ds(j * 16, 16)] = jnp.zeros((16,), jnp.float32)
        return 0
    lax.fori_loop(0, 40, zfill, 0)

    @pl.when(s < 10)
    def _():
        for k in range(25):
            pltpu.async_copy(zrow, acc_a.at[pl.ds(s * _RPT + k * 40, 40)], lsem0)
        for k in range(25):
            pltpu.make_async_copy(zrow, acc_a.at[pl.ds(s * _RPT + k * 40, 40)],
                                  lsem0).wait()
    plsc.subcore_barrier()

    def load(ci, b):
        eoff = (cstart + ci) * _CHUNK
        pltpu.async_copy(ei_h.at[:, pl.ds(eoff, _CHUNK)], eib[b], lsem[b])

    def wait_load(ci, b):
        eoff = (cstart + ci) * _CHUNK
        pltpu.make_async_copy(ei_h.at[:, pl.ds(eoff, _CHUNK)],
                              eib[b], lsem[b]).wait()

    def gather(b):
        pltpu.async_copy(feat_h.at[eib[b].at[0]], rows[b], gsem[b])

    def wait_gather(b):
        pltpu.make_async_copy(feat_h.at[eib[b].at[0]], rows[b], gsem[b]).wait()

    def scatter(b):
        for j in range(_CHUNK // 16):
            sl = pl.ds(j * 16, 16)
            didx[b][sl] = eib[b][1, sl]
        pltpu.async_copy(rows[b], acc_a.at[didx[b]], ssem[b], add=True)

    def drain(b):
        pltpu.make_async_copy(rows[b], acc_a.at[didx[b]], ssem[b]).wait()

    load(0, 0)
    load(1, 1)
    wait_load(0, 0)
    gather(0)

    def step(j, _):
        cB, cC, cD = 2 * j + 1, 2 * j + 2, 2 * j + 3
        @pl.when(j > 0)
        def _():
            drain(1)
        wait_load(cB, 1)
        gather(1)
        wait_gather(0)
        scatter(0)
        @pl.when(cC < nch)
        def _():
            load(cC, 0)
        drain(0)
        @pl.when(cC < nch)
        def _():
            wait_load(cC, 0)
            gather(0)
        wait_gather(1)
        scatter(1)
        @pl.when(cD < nch)
        def _():
            load(cD, 1)
        return 0
    lax.fori_loop(0, 39, step, 0)
    drain(1)

    @pl.when(nch == 79)
    def _():
        wait_gather(0)
        scatter(0)
        drain(0)
    plsc.subcore_barrier()

    @pl.when(s < 10)
    def _():
        pltpu.sync_copy(acc_a.at[pl.ds(s * _RPT, _RPT)],
                        a_o.at[c, pl.ds(s * _RPT, _RPT)])


_aggregate = pl.kernel(
    _aggregate_body,
    out_type=jax.ShapeDtypeStruct((_NC, _N, _DF), jnp.float32),
    mesh=_mesh,
    scratch_types=[
        pltpu.VMEM((2, _CHUNK), jnp.int32),
        pltpu.VMEM((2, _CHUNK), jnp.int32),
        pltpu.VMEM((_CHUNK,), jnp.int32),
        pltpu.VMEM((_CHUNK,), jnp.int32),
        pltpu.VMEM((_CHUNK, _DF), jnp.float32),
        pltpu.VMEM((_CHUNK, _DF), jnp.float32),
        pltpu.VMEM((40, _DF), jnp.float32),
        pltpu.VMEM_SHARED((_N, _DF), jnp.float32),
        pltpu.SemaphoreType.DMA,
        pltpu.SemaphoreType.DMA,
        pltpu.SemaphoreType.DMA,
        pltpu.SemaphoreType.DMA,
        pltpu.SemaphoreType.DMA,
        pltpu.SemaphoreType.DMA,
    ],
)


_BS = 2000
_NB = _N // _BS


def _scale_body(x_ref, c0_ref, c1_ref, feat_ref):
    deg = c0_ref[...] + c1_ref[...]
    ns = lax.rsqrt(jnp.maximum(deg, 1.0))
    feat_ref[...] = x_ref[...] * ns


def _scale(x, c0, c1):
    return pl.pallas_call(
        _scale_body,
        grid=(_NB,),
        in_specs=[pl.BlockSpec((_BS, _DF), lambda i: (i, 0)),
                  pl.BlockSpec((_BS, 1), lambda i: (i, 0)),
                  pl.BlockSpec((_BS, 1), lambda i: (2 * _N // _BS + i, 0))],
        out_specs=pl.BlockSpec((_BS, _DF), lambda i: (i, 0)),
        out_shape=jax.ShapeDtypeStruct((_N, _DF), jnp.float32),
    )(x, c0, c1)


def _final_body(a_ref, b_ref, w_ref, bias_ref, d0_ref, d1_ref, o_ref):
    a = a_ref[0] + a_ref[1]
    b = (b_ref[0] + b_ref[1])[:, :_DE]
    w = w_ref[...]
    h = jnp.dot(a, w[:_DF], preferred_element_type=jnp.float32,
                precision=lax.Precision.HIGHEST)
    h = h + jnp.dot(b, w[_DF:], preferred_element_type=jnp.float32,
                    precision=lax.Precision.HIGHEST)
    deg = d0_ref[...] + d1_ref[...]
    nd = lax.rsqrt(jnp.maximum(deg, 1.0))
    o_ref[...] = h * nd + bias_ref[...]


def _final(a, b, w, bias, d0, d1):
    return pl.pallas_call(
        _final_body,
        grid=(_NB,),
        in_specs=[pl.BlockSpec((_NC, _BS, _DF), lambda i: (0, i, 0)),
                  pl.BlockSpec((_NC, _BS, _DF), lambda i: (0, i, 0)),
                  pl.BlockSpec((_DF + _DE, _DO), lambda i: (0, 0)),
                  pl.BlockSpec((_DO,), lambda i: (0,)),
                  pl.BlockSpec((_BS, 1), lambda i: (_N // _BS + i, 0)),
                  pl.BlockSpec((_BS, 1), lambda i: (3 * _N // _BS + i, 0))],
        out_specs=pl.BlockSpec((_BS, _DO), lambda i: (i, 0)),
        out_shape=jax.ShapeDtypeStruct((_N, _DO), jnp.float32),
    )(a, b, w, bias, d0, d1)


def kernel(x, edge_index, edge_attr, weight, bias):
    ea_flat = edge_attr.reshape(-1)
    cnt, b_part = _edge_stats(edge_index, ea_flat)
    cnt4 = cnt.reshape(4 * _N, 1)
    feat = _scale(x, cnt4, cnt4)
    a_part = _aggregate(edge_index, feat)
    return _final(a_part, b_part, weight, bias, cnt4, cnt4)
